# baseline (speedup 1.0000x reference)
import functools

import jax
import jax.numpy as jnp
from jax import lax
from jax.experimental import pallas as pl
from jax.experimental.pallas import tpu as pltpu
from jax.experimental.pallas import tpu_sc as plsc


def _build(total, chunk):
    info = plsc.get_sparse_core_info()
    nw = info.num_cores * info.num_subcores
    b_per_w = total // nw
    n_chunks = b_per_w // chunk
    assert b_per_w * nw == total and n_chunks * chunk == b_per_w
    assert n_chunks % 2 == 0 and n_chunks >= 4

    mesh = plsc.VectorSubcoreMesh(core_axis_name="c", subcore_axis_name="s")

    buf_t = [
        pltpu.VMEM((chunk,), jnp.int32),
        pltpu.VMEM((chunk,), jnp.int32),
        pltpu.VMEM((chunk, 16), jnp.float32),
        pltpu.VMEM((chunk, 16), jnp.float32),
        pltpu.VMEM((chunk, 1), jnp.float32),
        pltpu.VMEM((chunk, 1), jnp.float32),
        pltpu.VMEM((chunk,), jnp.float32),
        pltpu.SemaphoreType.DMA,
        pltpu.SemaphoreType.DMA,
        pltpu.SemaphoreType.DMA,
        pltpu.SemaphoreType.DMA,
        pltpu.SemaphoreType.DMA,
    ]

    @functools.partial(
        pl.kernel,
        mesh=mesh,
        out_type=jax.ShapeDtypeStruct((total,), jnp.float32),
        compiler_params=pltpu.CompilerParams(
            needs_layout_passes=False, use_tc_tiling_on_sc=False
        ),
        scratch_types=buf_t + buf_t,
    )
    def fused_lookup(user_hbm, item_hbm, uf_hbm, pf_hbm, ub_hbm, pb_hbm,
                     out_hbm, *bufs):
        buf_a, buf_b = bufs[:12], bufs[12:]
        wid = lax.axis_index("s") * info.num_cores + lax.axis_index("c")
        base = wid * b_per_w
        lane = lax.iota(jnp.int32, 16)
        cols = [(lane + k) & 15 for k in range(16)]
        zero16 = lane & 0

        def fire_idx(buf, c):
            idx_u, idx_p = buf[0], buf[1]
            sem = buf[7]
            cbase = base + c * chunk
            pltpu.async_copy(user_hbm.at[pl.ds(cbase, chunk)], idx_u, sem)
            pltpu.async_copy(item_hbm.at[pl.ds(cbase, chunk)], idx_p, sem)

        def wait_idx(buf, c):
            idx_u, idx_p = buf[0], buf[1]
            sem = buf[7]
            cbase = base + c * chunk
            pltpu.make_async_copy(
                user_hbm.at[pl.ds(cbase, chunk)], idx_u, sem).wait()
            pltpu.make_async_copy(
                item_hbm.at[pl.ds(cbase, chunk)], idx_p, sem).wait()

        def fire_gathers(buf):
            idx_u, idx_p, urows, prows, ubv, pbv = buf[:6]
            pltpu.async_copy(uf_hbm.at[idx_u], urows, buf[8])
            pltpu.async_copy(pf_hbm.at[idx_p], prows, buf[9])
            pltpu.async_copy(ub_hbm.at[idx_u], ubv, buf[10])
            pltpu.async_copy(pb_hbm.at[idx_p], pbv, buf[11])

        def wait_gathers(buf):
            idx_u, idx_p, urows, prows, ubv, pbv = buf[:6]
            pltpu.make_async_copy(uf_hbm.at[idx_u], urows, buf[8]).wait()
            pltpu.make_async_copy(pf_hbm.at[idx_p], prows, buf[9]).wait()
            pltpu.make_async_copy(ub_hbm.at[idx_u], ubv, buf[10]).wait()
            pltpu.make_async_copy(pb_hbm.at[idx_p], pbv, buf[11]).wait()

        def compute(buf, c):
            urows, prows, ubv, pbv, outv = buf[2], buf[3], buf[4], buf[5], buf[6]
            cbase = base + c * chunk

            def group_body(g, _):
                g16 = g * 16
                row = g16 + lane
                acc = (plsc.load_gather(ubv, [row, zero16])
                       + plsc.load_gather(pbv, [row, zero16]))
                for k in range(16):
                    uc = plsc.load_gather(urows, [row, cols[k]])
                    pc = plsc.load_gather(prows, [row, cols[k]])
                    acc = acc + uc * pc
                outv[pl.ds(g16, 16)] = acc
                return 0

            lax.fori_loop(0, chunk // 16, group_body, 0)
            pltpu.sync_copy(outv, out_hbm.at[pl.ds(cbase, chunk)])

        fire_idx(buf_a, 0)
        wait_idx(buf_a, 0)
        fire_gathers(buf_a)
        fire_idx(buf_b, 1)

        def body2(h, _):
            c0 = h * 2
            wait_idx(buf_b, c0 + 1)
            fire_gathers(buf_b)
            wait_gathers(buf_a)
            fire_idx(buf_a, c0 + 2)
            compute(buf_a, c0)
            wait_idx(buf_a, c0 + 2)
            fire_gathers(buf_a)
            fire_idx(buf_b, c0 + 3)
            wait_gathers(buf_b)
            compute(buf_b, c0 + 1)
            return 0

        lax.fori_loop(0, n_chunks // 2 - 1, body2, 0)

        c0 = n_chunks - 2
        wait_idx(buf_b, c0 + 1)
        fire_gathers(buf_b)
        wait_gathers(buf_a)
        compute(buf_a, c0)
        wait_gathers(buf_b)
        compute(buf_b, c0 + 1)

    return fused_lookup


def kernel(user, item, user_factors, product_factors, user_bias, product_bias):
    b, l = user.shape
    total = b * l
    fused = _build(total, 1024)
    out = fused(
        user.T.reshape(total),
        item.T.reshape(total),
        user_factors,
        product_factors,
        user_bias,
        product_bias,
    )
    return out.reshape(l, b).T


# speedup vs baseline: 2.6225x; 2.6225x over previous
import functools

import jax
import jax.numpy as jnp
from jax import lax
from jax.experimental import pallas as pl
from jax.experimental.pallas import tpu as pltpu
from jax.experimental.pallas import tpu_sc as plsc


def _build(total, chunk):
    info = plsc.get_sparse_core_info()
    nw = info.num_cores * info.num_subcores
    b_per_w = total // nw
    n_chunks = b_per_w // chunk
    assert b_per_w * nw == total and n_chunks * chunk == b_per_w
    assert n_chunks % 2 == 0 and n_chunks >= 4

    mesh = plsc.VectorSubcoreMesh(core_axis_name="c", subcore_axis_name="s")

    buf_t = [
        pltpu.VMEM((chunk,), jnp.int32),
        pltpu.VMEM((chunk,), jnp.int32),
        pltpu.VMEM((chunk, 16), jnp.float32),
        pltpu.VMEM((chunk, 16), jnp.float32),
        pltpu.VMEM((chunk,), jnp.float32),
        pltpu.VMEM((chunk,), jnp.float32),
        pltpu.VMEM((chunk,), jnp.float32),
        pltpu.SemaphoreType.DMA,
        pltpu.SemaphoreType.DMA,
        pltpu.SemaphoreType.DMA,
        pltpu.SemaphoreType.DMA,
        pltpu.SemaphoreType.DMA,
    ]

    @functools.partial(
        pl.kernel,
        mesh=mesh,
        out_type=jax.ShapeDtypeStruct((total,), jnp.float32),
        compiler_params=pltpu.CompilerParams(
            needs_layout_passes=False, use_tc_tiling_on_sc=False
        ),
        scratch_types=buf_t + buf_t,
    )
    def fused_lookup(user_hbm, item_hbm, uf_hbm, pf_hbm, ub_hbm, pb_hbm,
                     out_hbm, *bufs):
        buf_a, buf_b = bufs[:12], bufs[12:]
        wid = lax.axis_index("s") * info.num_cores + lax.axis_index("c")
        base = wid * b_per_w
        lane = lax.iota(jnp.int32, 16)
        cols = [(lane + k) & 15 for k in range(16)]

        def fire_idx(buf, c):
            idx_u, idx_p = buf[0], buf[1]
            sem = buf[7]
            cbase = base + c * chunk
            pltpu.async_copy(user_hbm.at[pl.ds(cbase, chunk)], idx_u, sem)
            pltpu.async_copy(item_hbm.at[pl.ds(cbase, chunk)], idx_p, sem)

        def wait_idx(buf, c):
            idx_u, idx_p = buf[0], buf[1]
            sem = buf[7]
            cbase = base + c * chunk
            pltpu.make_async_copy(
                user_hbm.at[pl.ds(cbase, chunk)], idx_u, sem).wait()
            pltpu.make_async_copy(
                item_hbm.at[pl.ds(cbase, chunk)], idx_p, sem).wait()

        def fire_gathers(buf):
            idx_u, idx_p, urows, prows, ubv, pbv = buf[:6]
            pltpu.async_copy(uf_hbm.at[idx_u], urows, buf[8])
            pltpu.async_copy(pf_hbm.at[idx_p], prows, buf[9])
            pltpu.async_copy(ub_hbm.at[idx_u], ubv, buf[10])
            pltpu.async_copy(pb_hbm.at[idx_p], pbv, buf[11])

        def wait_gathers(buf):
            idx_u, idx_p, urows, prows, ubv, pbv = buf[:6]
            pltpu.make_async_copy(uf_hbm.at[idx_u], urows, buf[8]).wait()
            pltpu.make_async_copy(pf_hbm.at[idx_p], prows, buf[9]).wait()
            pltpu.make_async_copy(ub_hbm.at[idx_u], ubv, buf[10]).wait()
            pltpu.make_async_copy(pb_hbm.at[idx_p], pbv, buf[11]).wait()

        def compute(buf, c):
            urows, prows, ubv, pbv, outv = buf[2], buf[3], buf[4], buf[5], buf[6]
            cbase = base + c * chunk

            def group_body(g, _):
                g16 = g * 16
                row = g16 + lane
                acc = ubv[pl.ds(g16, 16)] + pbv[pl.ds(g16, 16)]
                for k in range(16):
                    uc = plsc.load_gather(urows, [row, cols[k]])
                    pc = plsc.load_gather(prows, [row, cols[k]])
                    acc = acc + uc * pc
                outv[pl.ds(g16, 16)] = acc
                return 0

            lax.fori_loop(0, chunk // 16, group_body, 0)
            pltpu.sync_copy(outv, out_hbm.at[pl.ds(cbase, chunk)])

        fire_idx(buf_a, 0)
        wait_idx(buf_a, 0)
        fire_gathers(buf_a)
        fire_idx(buf_b, 1)

        def body2(h, _):
            c0 = h * 2
            wait_idx(buf_b, c0 + 1)
            fire_gathers(buf_b)
            wait_gathers(buf_a)
            fire_idx(buf_a, c0 + 2)
            compute(buf_a, c0)
            wait_idx(buf_a, c0 + 2)
            fire_gathers(buf_a)
            fire_idx(buf_b, c0 + 3)
            wait_gathers(buf_b)
            compute(buf_b, c0 + 1)
            return 0

        lax.fori_loop(0, n_chunks // 2 - 1, body2, 0)

        c0 = n_chunks - 2
        wait_idx(buf_b, c0 + 1)
        fire_gathers(buf_b)
        wait_gathers(buf_a)
        compute(buf_a, c0)
        wait_gathers(buf_b)
        compute(buf_b, c0 + 1)

    return fused_lookup


def kernel(user, item, user_factors, product_factors, user_bias, product_bias):
    b, l = user.shape
    total = b * l
    fused = _build(total, 1280)
    out = fused(
        user.T.reshape(total),
        item.T.reshape(total),
        user_factors,
        product_factors,
        user_bias.T.reshape(-1),
        product_bias.T.reshape(-1),
    )
    return out.reshape(l, b).T


# speedup vs baseline: 2.6268x; 1.0017x over previous
import functools

import jax
import jax.numpy as jnp
from jax import lax
from jax.experimental import pallas as pl
from jax.experimental.pallas import tpu as pltpu
from jax.experimental.pallas import tpu_sc as plsc


def _build(total, chunk):
    info = plsc.get_sparse_core_info()
    nw = info.num_cores * info.num_subcores
    b_per_w = total // nw
    n_chunks = b_per_w // chunk
    assert b_per_w * nw == total and n_chunks * chunk == b_per_w
    assert n_chunks % 2 == 0 and n_chunks >= 4

    mesh = plsc.VectorSubcoreMesh(core_axis_name="c", subcore_axis_name="s")

    buf_t = [
        pltpu.VMEM((chunk,), jnp.int32),
        pltpu.VMEM((chunk,), jnp.int32),
        pltpu.VMEM((chunk, 16), jnp.float32),
        pltpu.VMEM((chunk, 16), jnp.float32),
        pltpu.VMEM((chunk,), jnp.float32),
        pltpu.VMEM((chunk,), jnp.float32),
        pltpu.VMEM((chunk,), jnp.float32),
        pltpu.SemaphoreType.DMA,
        pltpu.SemaphoreType.DMA,
        pltpu.SemaphoreType.DMA,
        pltpu.SemaphoreType.DMA,
        pltpu.SemaphoreType.DMA,
    ]

    @functools.partial(
        pl.kernel,
        mesh=mesh,
        out_type=jax.ShapeDtypeStruct((total,), jnp.float32),
        compiler_params=pltpu.CompilerParams(
            needs_layout_passes=False, use_tc_tiling_on_sc=False
        ),
        scratch_types=buf_t + buf_t,
    )
    def fused_lookup(user_hbm, item_hbm, uf_hbm, pf_hbm, ub_hbm, pb_hbm,
                     out_hbm, *bufs):
        buf_a, buf_b = bufs[:12], bufs[12:]
        wid = lax.axis_index("s") * info.num_cores + lax.axis_index("c")
        base = wid * b_per_w
        lane = lax.iota(jnp.int32, 16)
        cols = [(lane + k) & 15 for k in range(16)]

        def fire_idx(buf, c):
            idx_u, idx_p = buf[0], buf[1]
            sem = buf[7]
            cbase = base + c * chunk
            pltpu.async_copy(user_hbm.at[pl.ds(cbase, chunk)], idx_u, sem)
            pltpu.async_copy(item_hbm.at[pl.ds(cbase, chunk)], idx_p, sem)

        def wait_idx(buf, c):
            idx_u, idx_p = buf[0], buf[1]
            sem = buf[7]
            cbase = base + c * chunk
            pltpu.make_async_copy(
                user_hbm.at[pl.ds(cbase, chunk)], idx_u, sem).wait()
            pltpu.make_async_copy(
                item_hbm.at[pl.ds(cbase, chunk)], idx_p, sem).wait()

        def fire_gathers(buf):
            idx_u, idx_p, urows, prows, ubv, pbv = buf[:6]
            pltpu.async_copy(uf_hbm.at[idx_u], urows, buf[8])
            pltpu.async_copy(pf_hbm.at[idx_p], prows, buf[9])
            pltpu.async_copy(ub_hbm.at[idx_u], ubv, buf[10])
            pltpu.async_copy(pb_hbm.at[idx_p], pbv, buf[11])

        def wait_gathers(buf):
            idx_u, idx_p, urows, prows, ubv, pbv = buf[:6]
            pltpu.make_async_copy(uf_hbm.at[idx_u], urows, buf[8]).wait()
            pltpu.make_async_copy(pf_hbm.at[idx_p], prows, buf[9]).wait()
            pltpu.make_async_copy(ub_hbm.at[idx_u], ubv, buf[10]).wait()
            pltpu.make_async_copy(pb_hbm.at[idx_p], pbv, buf[11]).wait()

        def compute(buf, c):
            urows, prows, ubv, pbv, outv = buf[2], buf[3], buf[4], buf[5], buf[6]
            cbase = base + c * chunk

            def group_body(g, _):
                g16 = g * 16
                row = g16 + lane
                acc = ubv[pl.ds(g16, 16)] + pbv[pl.ds(g16, 16)]
                for k in range(16):
                    uc = plsc.load_gather(urows, [row, cols[k]])
                    pc = plsc.load_gather(prows, [row, cols[k]])
                    acc = acc + uc * pc
                outv[pl.ds(g16, 16)] = acc
                return 0

            lax.fori_loop(0, chunk // 16, group_body, 0)
            pltpu.sync_copy(outv, out_hbm.at[pl.ds(cbase, chunk)])

        fire_idx(buf_a, 0)
        wait_idx(buf_a, 0)
        fire_gathers(buf_a)
        fire_idx(buf_b, 1)

        def body2(h, _):
            c0 = h * 2
            wait_idx(buf_b, c0 + 1)
            fire_gathers(buf_b)
            wait_gathers(buf_a)
            fire_idx(buf_a, c0 + 2)
            compute(buf_a, c0)
            wait_idx(buf_a, c0 + 2)
            fire_gathers(buf_a)
            fire_idx(buf_b, c0 + 3)
            wait_gathers(buf_b)
            compute(buf_b, c0 + 1)
            return 0

        lax.fori_loop(0, n_chunks // 2 - 1, body2, 0)

        c0 = n_chunks - 2
        wait_idx(buf_b, c0 + 1)
        fire_gathers(buf_b)
        wait_gathers(buf_a)
        compute(buf_a, c0)
        wait_gathers(buf_b)
        compute(buf_b, c0 + 1)

    return fused_lookup


def kernel(user, item, user_factors, product_factors, user_bias, product_bias):
    b, l = user.shape
    total = b * l
    fused = _build(total, 1024)
    out = fused(
        user.T.reshape(total),
        item.T.reshape(total),
        user_factors,
        product_factors,
        user_bias.T.reshape(-1),
        product_bias.T.reshape(-1),
    )
    return out.reshape(l, b).T
